# Initial kernel scaffold; baseline (speedup 1.0000x reference)
import functools

import jax
import jax.numpy as jnp
from jax import lax
from jax.experimental import pallas as pl
from jax.experimental.pallas import tpu as pltpu
from jax.experimental.pallas import tpu_sc as plsc

V, D = 1_000_000, 32
B, L = 16384, 50
NTOT = B * L

NC, NS = 2, 16
NW = NC * NS
PER_W = NTOT // NW
G = 8
CHUNK = G * 128
NCHUNK = PER_W // CHUNK


def _sc_gather(idx_flat, table):
    mesh = plsc.VectorSubcoreMesh(core_axis_name="c", subcore_axis_name="s")

    @functools.partial(
        pl.kernel,
        out_type=jax.ShapeDtypeStruct((NTOT, D), jnp.float32),
        mesh=mesh,
        scratch_types=[
            pltpu.VMEM((G, 128), jnp.int32),
            pltpu.VMEM((G, 128, D), jnp.float32),
            pltpu.SemaphoreType.DMA,
        ],
    )
    def k(idx_hbm, table_hbm, out_hbm, idx_v, rows_v, sem):
        wid = lax.axis_index("s") * NC + lax.axis_index("c")
        wbase = wid * PER_W

        def body(i, _):
            base = wbase + i * CHUNK
            pltpu.sync_copy(idx_hbm.at[pl.ds(base, CHUNK)].reshape(G, 128),
                            idx_v)
            copies = [
                pltpu.async_copy(table_hbm.at[idx_v.at[j]], rows_v.at[j], sem)
                for j in range(G)
            ]
            for c in copies:
                c.wait()
            for j in range(G):
                pltpu.sync_copy(rows_v.at[j],
                                out_hbm.at[pl.ds(base + j * 128, 128)])
            return ()

        lax.fori_loop(0, NCHUNK, body, (), unroll=False)

    return k(idx_flat, table)


def kernel(idx, table):
    return _sc_gather(idx.reshape(-1), table)


# speedup vs baseline: 1.0434x; 1.0434x over previous
import functools

import jax
import jax.numpy as jnp
from jax import lax
from jax.experimental import pallas as pl
from jax.experimental.pallas import tpu as pltpu
from jax.experimental.pallas import tpu_sc as plsc

V, D = 1_000_000, 32
B, L = 16384, 50
NTOT = B * L

NC, NS = 2, 16
NW = NC * NS
PER_W = NTOT // NW
G = 8
CHUNK = G * 128
NCHUNK = PER_W // CHUNK


def _sc_gather(idx_flat, table):
    mesh = plsc.VectorSubcoreMesh(core_axis_name="c", subcore_axis_name="s")

    @functools.partial(
        pl.kernel,
        out_type=jax.ShapeDtypeStruct((NTOT, D), jnp.float32),
        mesh=mesh,
        scratch_types=[
            pltpu.VMEM((CHUNK,), jnp.int32),
            pltpu.VMEM((G, 128, D), jnp.float32),
            pltpu.SemaphoreType.DMA,
        ],
        compiler_params=pltpu.CompilerParams(use_tc_tiling_on_sc=False),
    )
    def k(idx_hbm, table_hbm, out_hbm, idx_v, rows_v, sem):
        wid = lax.axis_index("s") * NC + lax.axis_index("c")
        wbase = wid * PER_W

        def body(i, _):
            base = wbase + i * CHUNK
            pltpu.sync_copy(idx_hbm.at[pl.ds(base, CHUNK)], idx_v)
            copies = [
                pltpu.async_copy(
                    table_hbm.at[idx_v.at[pl.ds(j * 128, 128)]],
                    rows_v.at[j], sem)
                for j in range(G)
            ]
            for c in copies:
                c.wait()
            for j in range(G):
                pltpu.sync_copy(rows_v.at[j],
                                out_hbm.at[pl.ds(base + j * 128, 128)])
            return ()

        lax.fori_loop(0, NCHUNK, body, (), unroll=False)

    return k(idx_flat, table)


def kernel(idx, table):
    return _sc_gather(idx.reshape(-1), table)


# speedup vs baseline: 1.0749x; 1.0302x over previous
import functools

import jax
import jax.numpy as jnp
from jax import lax
from jax.experimental import pallas as pl
from jax.experimental.pallas import tpu as pltpu
from jax.experimental.pallas import tpu_sc as plsc

V, D = 1_000_000, 32
B, L = 16384, 50
NTOT = B * L

NC, NS = 2, 16
NW = NC * NS
PER_W = NTOT // NW
G = 10
CHUNK = G * 128
NCHUNK = PER_W // CHUNK


def _sc_gather(idx_flat, table):
    mesh = plsc.VectorSubcoreMesh(core_axis_name="c", subcore_axis_name="s")

    @functools.partial(
        pl.kernel,
        out_type=jax.ShapeDtypeStruct((NTOT, D), jnp.float32),
        mesh=mesh,
        scratch_types=[
            pltpu.VMEM((2, CHUNK), jnp.int32),
            pltpu.VMEM((2, CHUNK, D), jnp.float32),
            pltpu.SemaphoreType.DMA,
            pltpu.SemaphoreType.DMA,
            pltpu.SemaphoreType.DMA,
            pltpu.SemaphoreType.DMA,
        ],
        compiler_params=pltpu.CompilerParams(use_tc_tiling_on_sc=False),
    )
    def k(idx_hbm, table_hbm, out_hbm, idx_v, rows_v,
          gsem0, gsem1, osem0, osem1):
        gsem = (gsem0, gsem1)
        osem = (osem0, osem1)
        wid = lax.axis_index("s") * NC + lax.axis_index("c")
        wbase = wid * PER_W

        def fire_gathers(buf, chunk):
            base = wbase + chunk * CHUNK
            pltpu.sync_copy(idx_hbm.at[pl.ds(base, CHUNK)], idx_v.at[buf])
            for j in range(G):
                pltpu.async_copy(
                    table_hbm.at[idx_v.at[buf].at[pl.ds(j * 128, 128)]],
                    rows_v.at[buf].at[pl.ds(j * 128, 128)], gsem[buf])

        def wait_gathers(buf):
            for j in range(G):
                pltpu.make_async_copy(
                    table_hbm.at[idx_v.at[buf].at[pl.ds(j * 128, 128)]],
                    rows_v.at[buf].at[pl.ds(j * 128, 128)], gsem[buf]).wait()

        def fire_writeback(buf, chunk):
            base = wbase + chunk * CHUNK
            pltpu.async_copy(rows_v.at[buf], out_hbm.at[pl.ds(base, CHUNK)],
                             osem[buf])

        def wait_writeback(buf, chunk):
            base = wbase + chunk * CHUNK
            pltpu.make_async_copy(rows_v.at[buf],
                                  out_hbm.at[pl.ds(base, CHUNK)],
                                  osem[buf]).wait()

        fire_gathers(0, 0)

        def body(i2, _):
            for b in range(2):
                i = 2 * i2 + b
                nbuf = 1 - b

                @pl.when(i >= 1)
                def _():
                    wait_writeback(nbuf, i - 1)

                @pl.when(i + 1 < NCHUNK)
                def _():
                    fire_gathers(nbuf, i + 1)

                wait_gathers(b)
                fire_writeback(b, i)
            return ()

        lax.fori_loop(0, NCHUNK // 2, body, (), unroll=False)
        wait_writeback((NCHUNK - 1) % 2, NCHUNK - 1)

    return k(idx_flat, table)


def kernel(idx, table):
    return _sc_gather(idx.reshape(-1), table)
